# baseline (speedup 1.0000x reference)
import functools

import jax
import jax.numpy as jnp
from jax import lax
from jax.experimental import pallas as pl
from jax.experimental.pallas import tpu as pltpu
from jax.experimental.pallas import tpu_sc as plsc

F32 = jnp.float32
BF16 = jnp.bfloat16
I32 = jnp.int32
_HIGH = lax.Precision.HIGHEST
BLK = 256
CH = 512


def _route_body(x_ref, gw_ref, pos_ref, warr_ref, inv1_ref, inv2_ref,
                blkexp_ref, nblk_ref):
    x = x_ref[...]
    gw = gw_ref[...]
    T = x.shape[0]
    T2 = 2 * T
    logits = lax.dot_general(x, gw, (((1,), (1,)), ((), ())),
                             preferred_element_type=F32)
    lane = lax.broadcasted_iota(I32, (T, 128), 1)
    logits = jnp.where(lane < 8, logits, -1e30)
    m = jnp.max(logits, axis=1, keepdims=True)
    p = jnp.exp(logits - m)
    p = p / jnp.sum(p, axis=1, keepdims=True)
    w1 = jnp.max(p, axis=1, keepdims=True)
    i1 = jnp.min(jnp.where(p == w1, lane, 999), axis=1, keepdims=True)
    p2 = jnp.where(lane == i1, -1.0, p)
    w2 = jnp.max(p2, axis=1, keepdims=True)
    i2 = jnp.min(jnp.where(p2 == w2, lane, 999), axis=1, keepdims=True)

    e_arr = jnp.concatenate([i1, i2], axis=0)
    w_arr = jnp.concatenate([w1, w2], axis=0)
    lane2 = lax.broadcasted_iota(I32, (T2, 128), 1)
    A = (e_arr == lane2).astype(F32)
    counts = jnp.sum(A, axis=0, keepdims=True)
    cnt_pad = jnp.floor((counts + (BLK - 1)) * (1.0 / BLK)) * BLK
    r128 = lax.broadcasted_iota(I32, (128, 128), 0)
    c128 = lax.broadcasted_iota(I32, (128, 128), 1)
    tri_excl = (r128 < c128).astype(F32)
    offsets = lax.dot_general(cnt_pad, tri_excl, (((1,), (0,)), ((), ())),
                              precision=_HIGH, preferred_element_type=F32)
    rS = lax.broadcasted_iota(I32, (CH, CH), 0)
    cS = lax.broadcasted_iota(I32, (CH, CH), 1)
    tri_strict = (rS > cS).astype(F32)
    run = jnp.zeros((1, 128), F32)
    pos_list = []
    for t in range(T2 // CH):
        At = A[t * CH:(t + 1) * CH]
        within = lax.dot_general(tri_strict, At, (((1,), (0,)), ((), ())),
                                 precision=_HIGH, preferred_element_type=F32)
        pos_list.append(
            jnp.sum(At * (within + run + offsets), axis=1, keepdims=True))
        run = run + jnp.sum(At, axis=0, keepdims=True)
    pos = jnp.concatenate(pos_list, axis=0)
    inv1_ref[...] = pos[:T].astype(I32)
    inv2_ref[...] = pos[T:].astype(I32)

    pos_ref[...] = pos
    warr_ref[...] = w_arr

    MB = blkexp_ref.shape[0]
    rowstart = lax.broadcasted_iota(I32, (MB, 128), 0).astype(F32) * BLK
    indb = ((rowstart >= offsets) & (rowstart < offsets + cnt_pad)
            & (cnt_pad > 0)).astype(F32)
    lanef = lax.broadcasted_iota(I32, (MB, 128), 1).astype(F32)
    be = jnp.sum(indb * lanef, axis=1, keepdims=True)
    validb = jnp.sum(indb, axis=1, keepdims=True)
    lastexp = jnp.max(jnp.where(counts > 0,
                                lax.broadcasted_iota(I32, (1, 128), 1).astype(F32), 0.0),
                      axis=1, keepdims=True)
    blkexp_ref[...] = jnp.where(validb > 0, be, lastexp).astype(I32)
    nblk_ref[...] = (jnp.sum(cnt_pad, axis=1, keepdims=True)
                     * (1.0 / BLK)).astype(I32)


def _scatter_body(pos_ref, warr_ref, tok_ref, perm_ref, wsort_ref):
    q = pl.program_id(0)
    pos = pos_ref[...]
    T2 = pos.shape[0]
    pv = lax.broadcasted_iota(I32, (T2, CH), 1).astype(F32) + (q * CH).astype(
        F32)
    ind = (pos == pv).astype(F32)
    perm_ref[0] = jnp.sum(ind * tok_ref[...], axis=0,
                          keepdims=True).astype(I32)
    wsort_ref[0] = jnp.sum(ind * warr_ref[...], axis=0, keepdims=True)


def _group_body(be_ref, nb_ref, xs_ref, ws_ref, wg_ref, wu_ref, wd_ref,
                out_ref):
    b = pl.program_id(0)

    @pl.when(b < nb_ref[0])
    def _():
        xb = xs_ref[...]
        g = lax.dot_general(xb, wg_ref[0], (((1,), (1,)), ((), ())),
                            preferred_element_type=F32)
        u = lax.dot_general(xb, wu_ref[0], (((1,), (1,)), ((), ())),
                            preferred_element_type=F32)
        h = (g * jax.nn.sigmoid(g) * u).astype(BF16)
        y = lax.dot_general(h, wd_ref[0], (((1,), (1,)), ((), ())),
                            preferred_element_type=F32)
        out_ref[...] = y * ws_ref[...]


def _shared_body(x_ref, sg_ref, su_ref, sd_ref, segw_ref, out_ref):
    j = pl.program_id(1)
    nj = pl.num_programs(1)
    x = x_ref[...]
    g = lax.dot_general(x, sg_ref[...], (((1,), (1,)), ((), ())),
                        preferred_element_type=F32)
    u = lax.dot_general(x, su_ref[...], (((1,), (1,)), ((), ())),
                        preferred_element_type=F32)
    h = (g * jax.nn.sigmoid(g) * u).astype(BF16)
    y = lax.dot_general(h, sd_ref[...], (((1,), (1,)), ((), ())),
                        preferred_element_type=F32)

    @pl.when(j == 0)
    def _init():
        out_ref[...] = y

    @pl.when(j > 0)
    def _acc():
        out_ref[...] += y

    @pl.when(j == nj - 1)
    def _fin():
        sl = lax.dot_general(x, segw_ref[...], (((1,), (1,)), ((), ())),
                             preferred_element_type=F32)
        lane = lax.broadcasted_iota(I32, sl.shape, 1)
        gate = jnp.sum(jnp.where(lane == 0, jax.nn.sigmoid(sl), 0.0),
                       axis=1, keepdims=True)
        out_ref[...] = gate * out_ref[...]


def _sc_gather(x3, perm, NP):
    Tn, W = x3.shape
    NW = 32
    rows_per = NP // NW
    CG = rows_per // 2 if rows_per > 96 else rows_per
    mesh = plsc.VectorSubcoreMesh(core_axis_name="c", subcore_axis_name="s")

    @functools.partial(
        pl.kernel, mesh=mesh,
        out_type=jax.ShapeDtypeStruct((NP, W), I32),
        scratch_types=[
            pltpu.VMEM((CG,), I32),
            pltpu.VMEM((CG, W), I32),
            pltpu.SemaphoreType.DMA,
        ],
    )
    def gk(x_hbm, perm_hbm, out_hbm, idx_v, rows_v, sem):
        wid = lax.axis_index("s") * 2 + lax.axis_index("c")
        for c in range(rows_per // CG):
            base = wid * rows_per + c * CG
            pltpu.sync_copy(perm_hbm.at[pl.ds(base, CG)], idx_v)
            pltpu.async_copy(x_hbm.at[idx_v], rows_v, sem).wait()
            pltpu.sync_copy(rows_v, out_hbm.at[pl.ds(base, CG)])

    return gk(x3, perm)


def _sc_combine(shared, ys, inv1, inv2):
    T, D = shared.shape
    NW = 32
    per = T // NW
    CT = 16
    mesh = plsc.VectorSubcoreMesh(core_axis_name="c", subcore_axis_name="s")

    @functools.partial(
        pl.kernel, mesh=mesh,
        out_type=jax.ShapeDtypeStruct((T, D), F32),
        scratch_types=[
            pltpu.VMEM((CT,), I32),
            pltpu.VMEM((CT,), I32),
            pltpu.VMEM((CT, D), F32),
            pltpu.VMEM((CT, D), F32),
            pltpu.VMEM((CT, D), F32),
            pltpu.SemaphoreType.DMA,
            pltpu.SemaphoreType.DMA,
        ],
    )
    def ck(sh_hbm, ys_hbm, i1_hbm, i2_hbm, out_hbm, x1_v, x2_v, b0, b1, b2,
           sem1, sem2):
        wid = lax.axis_index("s") * 2 + lax.axis_index("c")
        for c in range(per // CT):
            base = wid * per + c * CT
            pltpu.sync_copy(i1_hbm.at[pl.ds(base, CT)], x1_v)
            pltpu.sync_copy(i2_hbm.at[pl.ds(base, CT)], x2_v)
            cp1 = pltpu.async_copy(ys_hbm.at[x1_v], b1, sem1)
            cp2 = pltpu.async_copy(ys_hbm.at[x2_v], b2, sem2)
            pltpu.sync_copy(sh_hbm.at[pl.ds(base, CT)], b0)
            cp1.wait()
            cp2.wait()

            def cbody(i, carry):
                for r in range(CT):
                    s = pl.ds(i * 16, 16)
                    b0[r, s] = b0[r, s] + b1[r, s] + b2[r, s]
                return carry

            lax.fori_loop(0, D // 16, cbody, 0)
            pltpu.sync_copy(b0, out_hbm.at[pl.ds(base, CT)])

    return ck(shared, ys, inv1, inv2)


def kernel(hidden_states, gate_w, Wg, Wu, Wd, Sg, Su, Sd, seg_w):
    b, s, d = hidden_states.shape
    x = hidden_states.reshape(-1, d)
    T, D = x.shape
    E, DFF, _ = Wg.shape
    DFF_S = Sg.shape[0]
    NP = ((2 * T + E * (BLK - 1)) + CH - 1) // CH * CH
    MAXB = NP // BLK
    MB = (MAXB + 7) // 8 * 8

    gw_pad = jnp.zeros((128, D), F32).at[:E].set(gate_w)
    pos, warr, inv1, inv2, blkexp, nblk = pl.pallas_call(
        _route_body,
        out_shape=[
            jax.ShapeDtypeStruct((2 * T, 1), F32),
            jax.ShapeDtypeStruct((2 * T, 1), F32),
            jax.ShapeDtypeStruct((T, 1), I32),
            jax.ShapeDtypeStruct((T, 1), I32),
            jax.ShapeDtypeStruct((MB, 1), I32),
            jax.ShapeDtypeStruct((1, 1), I32),
        ],
    )(x, gw_pad)

    tok = jnp.tile(jnp.arange(T, dtype=F32), 2).reshape(2 * T, 1)
    perm12, wsort12 = pl.pallas_call(
        _scatter_body,
        grid=(NP // CH,),
        in_specs=[
            pl.BlockSpec((2 * T, 1), lambda q: (0, 0)),
            pl.BlockSpec((2 * T, 1), lambda q: (0, 0)),
            pl.BlockSpec((2 * T, 1), lambda q: (0, 0)),
        ],
        out_specs=[
            pl.BlockSpec((1, 1, CH), lambda q: (q, 0, 0)),
            pl.BlockSpec((1, 1, CH), lambda q: (q, 0, 0)),
        ],
        out_shape=[
            jax.ShapeDtypeStruct((NP // CH, 1, CH), I32),
            jax.ShapeDtypeStruct((NP // CH, 1, CH), F32),
        ],
    )(pos, warr, tok)

    xbf = x.astype(BF16)
    xi = lax.bitcast_convert_type(xbf.reshape(T, D // 2, 2), I32)
    xsi = _sc_gather(xi, perm12.reshape(NP), NP)
    xs = lax.bitcast_convert_type(xsi, BF16).reshape(NP, D)

    ys = pl.pallas_call(
        _group_body,
        grid_spec=pltpu.PrefetchScalarGridSpec(
            num_scalar_prefetch=2,
            grid=(MAXB,),
            in_specs=[
                pl.BlockSpec((BLK, D), lambda bb, be, nb: (bb, 0)),
                pl.BlockSpec((BLK, 1), lambda bb, be, nb: (bb, 0)),
                pl.BlockSpec((1, DFF, D), lambda bb, be, nb: (be[bb], 0, 0)),
                pl.BlockSpec((1, DFF, D), lambda bb, be, nb: (be[bb], 0, 0)),
                pl.BlockSpec((1, D, DFF), lambda bb, be, nb: (be[bb], 0, 0)),
            ],
            out_specs=pl.BlockSpec((BLK, D), lambda bb, be, nb: (bb, 0)),
        ),
        out_shape=jax.ShapeDtypeStruct((NP, D), F32),
    )(blkexp.reshape(MB), nblk.reshape(1), xs, wsort12.reshape(NP, 1),
      Wg.astype(BF16), Wu.astype(BF16), Wd.astype(BF16))

    BT = min(512, T)
    BF = 512 if DFF_S % 512 == 0 else DFF_S
    segw_pad = jnp.zeros((128, D), BF16).at[:1].set(seg_w.astype(BF16))
    shared = pl.pallas_call(
        _shared_body,
        grid=(T // BT, DFF_S // BF),
        in_specs=[
            pl.BlockSpec((BT, D), lambda i, j: (i, 0)),
            pl.BlockSpec((BF, D), lambda i, j: (j, 0)),
            pl.BlockSpec((BF, D), lambda i, j: (j, 0)),
            pl.BlockSpec((D, BF), lambda i, j: (0, j)),
            pl.BlockSpec((128, D), lambda i, j: (0, 0)),
        ],
        out_specs=pl.BlockSpec((BT, D), lambda i, j: (i, 0)),
        out_shape=jax.ShapeDtypeStruct((T, D), F32),
    )(xbf, Sg.astype(BF16), Su.astype(BF16), Sd.astype(BF16), segw_pad)

    out = _sc_combine(shared, ys, inv1.reshape(T), inv2.reshape(T))
    return out.reshape(b, s, d)


# speedup vs baseline: 1.4570x; 1.4570x over previous
import functools

import jax
import jax.numpy as jnp
from jax import lax
from jax.experimental import pallas as pl
from jax.experimental.pallas import tpu as pltpu
from jax.experimental.pallas import tpu_sc as plsc

F32 = jnp.float32
BF16 = jnp.bfloat16
I32 = jnp.int32
_HIGH = lax.Precision.HIGHEST
BLK = 256
CH = 512


def _route_body(x_ref, gw_ref, pos_ref, warr_ref, inv1_ref, inv2_ref,
                blkexp_ref, nblk_ref, xi_ref):
    x = x_ref[...]
    gw = gw_ref[...]
    T = x.shape[0]
    T2 = 2 * T
    D = x.shape[1]
    U32 = jnp.uint32

    def _rne16(v):
        bb = lax.bitcast_convert_type(v, U32)
        return (bb + U32(0x7FFF) + ((bb >> U32(16)) & U32(1))) >> U32(16)

    lo = _rne16(x[:, :D // 2])
    hi = _rne16(x[:, D // 2:])
    xi_ref[...] = lax.bitcast_convert_type(lo | (hi << U32(16)), I32)
    logits = lax.dot_general(x, gw, (((1,), (1,)), ((), ())),
                             preferred_element_type=F32)
    lane = lax.broadcasted_iota(I32, (T, 128), 1)
    logits = jnp.where(lane < 8, logits, -1e30)
    m = jnp.max(logits, axis=1, keepdims=True)
    p = jnp.exp(logits - m)
    p = p / jnp.sum(p, axis=1, keepdims=True)
    w1 = jnp.max(p, axis=1, keepdims=True)
    i1 = jnp.min(jnp.where(p == w1, lane, 999), axis=1, keepdims=True)
    p2 = jnp.where(lane == i1, -1.0, p)
    w2 = jnp.max(p2, axis=1, keepdims=True)
    i2 = jnp.min(jnp.where(p2 == w2, lane, 999), axis=1, keepdims=True)

    e_arr = jnp.concatenate([i1, i2], axis=0)
    w_arr = jnp.concatenate([w1, w2], axis=0)
    lane2 = lax.broadcasted_iota(I32, (T2, 128), 1)
    A = (e_arr == lane2).astype(F32)
    counts = jnp.sum(A, axis=0, keepdims=True)
    cnt_pad = jnp.floor((counts + (BLK - 1)) * (1.0 / BLK)) * BLK
    r128 = lax.broadcasted_iota(I32, (128, 128), 0)
    c128 = lax.broadcasted_iota(I32, (128, 128), 1)
    tri_excl = (r128 < c128).astype(F32)
    offsets = lax.dot_general(cnt_pad, tri_excl, (((1,), (0,)), ((), ())),
                              precision=_HIGH, preferred_element_type=F32)
    rS = lax.broadcasted_iota(I32, (CH, CH), 0)
    cS = lax.broadcasted_iota(I32, (CH, CH), 1)
    tri_strict = (rS > cS).astype(F32)
    run = jnp.zeros((1, 128), F32)
    pos_list = []
    for t in range(T2 // CH):
        At = A[t * CH:(t + 1) * CH]
        within = lax.dot_general(tri_strict, At, (((1,), (0,)), ((), ())),
                                 precision=_HIGH, preferred_element_type=F32)
        pos_list.append(
            jnp.sum(At * (within + run + offsets), axis=1, keepdims=True))
        run = run + jnp.sum(At, axis=0, keepdims=True)
    pos = jnp.concatenate(pos_list, axis=0)
    inv1_ref[...] = pos[:T].astype(I32)
    inv2_ref[...] = pos[T:].astype(I32)

    pos_ref[...] = pos
    warr_ref[...] = w_arr

    MB = blkexp_ref.shape[0]
    rowstart = lax.broadcasted_iota(I32, (MB, 128), 0).astype(F32) * BLK
    indb = ((rowstart >= offsets) & (rowstart < offsets + cnt_pad)
            & (cnt_pad > 0)).astype(F32)
    lanef = lax.broadcasted_iota(I32, (MB, 128), 1).astype(F32)
    be = jnp.sum(indb * lanef, axis=1, keepdims=True)
    validb = jnp.sum(indb, axis=1, keepdims=True)
    lastexp = jnp.max(jnp.where(counts > 0,
                                lax.broadcasted_iota(I32, (1, 128), 1).astype(F32), 0.0),
                      axis=1, keepdims=True)
    blkexp_ref[...] = jnp.where(validb > 0, be, lastexp).astype(I32)
    nblk_ref[...] = (jnp.sum(cnt_pad, axis=1, keepdims=True)
                     * (1.0 / BLK)).astype(I32)


def _scatter_body(pos_ref, warr_ref, tok_ref, perm_ref, wsort_ref):
    q = pl.program_id(0)
    pos = pos_ref[...]
    T2 = pos.shape[0]
    pv = lax.broadcasted_iota(I32, (T2, CH), 1).astype(F32) + (q * CH).astype(
        F32)
    ind = (pos == pv).astype(F32)
    perm_ref[0] = jnp.sum(ind * tok_ref[...], axis=0,
                          keepdims=True).astype(I32)
    wsort_ref[0] = jnp.sum(ind * warr_ref[...], axis=0, keepdims=True)


def _group_body(be_ref, nb_ref, xs_ref, ws_ref, wg_ref, wu_ref, wd_ref,
                out_ref):
    b = pl.program_id(0)

    @pl.when(b < nb_ref[0])
    def _():
        xw = lax.bitcast_convert_type(xs_ref[...], jnp.uint32)
        xlo = lax.bitcast_convert_type(xw << jnp.uint32(16), F32).astype(BF16)
        xhi = lax.bitcast_convert_type(xw & jnp.uint32(0xFFFF0000),
                                       F32).astype(BF16)
        xb = jnp.concatenate([xlo, xhi], axis=1)
        g = lax.dot_general(xb, wg_ref[0], (((1,), (1,)), ((), ())),
                            preferred_element_type=F32)
        u = lax.dot_general(xb, wu_ref[0], (((1,), (1,)), ((), ())),
                            preferred_element_type=F32)
        h = (g * jax.nn.sigmoid(g) * u).astype(BF16)
        y = lax.dot_general(h, wd_ref[0], (((1,), (1,)), ((), ())),
                            preferred_element_type=F32)
        out_ref[...] = y * ws_ref[...]


def _shared_body(x_ref, sg_ref, su_ref, sd_ref, segw_ref, out_ref):
    j = pl.program_id(1)
    nj = pl.num_programs(1)
    x = x_ref[...].astype(BF16)
    g = lax.dot_general(x, sg_ref[...], (((1,), (1,)), ((), ())),
                        preferred_element_type=F32)
    u = lax.dot_general(x, su_ref[...], (((1,), (1,)), ((), ())),
                        preferred_element_type=F32)
    h = (g * jax.nn.sigmoid(g) * u).astype(BF16)
    y = lax.dot_general(h, sd_ref[...], (((1,), (1,)), ((), ())),
                        preferred_element_type=F32)

    @pl.when(j == 0)
    def _init():
        out_ref[...] = y

    @pl.when(j > 0)
    def _acc():
        out_ref[...] += y

    @pl.when(j == nj - 1)
    def _fin():
        sl = lax.dot_general(x, segw_ref[...], (((1,), (1,)), ((), ())),
                             preferred_element_type=F32)
        lane = lax.broadcasted_iota(I32, sl.shape, 1)
        gate = jnp.sum(jnp.where(lane == 0, jax.nn.sigmoid(sl), 0.0),
                       axis=1, keepdims=True)
        out_ref[...] = gate * out_ref[...]


def _sc_gather(x3, perm, NP):
    Tn, W = x3.shape
    NW = 32
    rows_per = NP // NW
    CG = rows_per // 2 if rows_per > 96 else rows_per
    mesh = plsc.VectorSubcoreMesh(core_axis_name="c", subcore_axis_name="s")

    @functools.partial(
        pl.kernel, mesh=mesh,
        out_type=jax.ShapeDtypeStruct((NP, W), I32),
        scratch_types=[
            pltpu.VMEM((CG,), I32),
            pltpu.VMEM((CG, W), I32),
            pltpu.SemaphoreType.DMA,
        ],
    )
    def gk(x_hbm, perm_hbm, out_hbm, idx_v, rows_v, sem):
        wid = lax.axis_index("s") * 2 + lax.axis_index("c")
        for c in range(rows_per // CG):
            base = wid * rows_per + c * CG
            pltpu.sync_copy(perm_hbm.at[pl.ds(base, CG)], idx_v)
            pltpu.async_copy(x_hbm.at[idx_v], rows_v, sem).wait()
            pltpu.sync_copy(rows_v, out_hbm.at[pl.ds(base, CG)])

    return gk(x3, perm)


def _sc_combine(shared, ys, inv1, inv2):
    T, D = shared.shape
    NW = 32
    per = T // NW
    CT = 16
    mesh = plsc.VectorSubcoreMesh(core_axis_name="c", subcore_axis_name="s")

    @functools.partial(
        pl.kernel, mesh=mesh,
        out_type=jax.ShapeDtypeStruct((T, D), F32),
        scratch_types=[
            pltpu.VMEM((CT,), I32),
            pltpu.VMEM((CT,), I32),
            pltpu.VMEM((CT, D), F32),
            pltpu.VMEM((CT, D), F32),
            pltpu.VMEM((CT, D), F32),
            pltpu.SemaphoreType.DMA,
            pltpu.SemaphoreType.DMA,
        ],
    )
    def ck(sh_hbm, ys_hbm, i1_hbm, i2_hbm, out_hbm, x1_v, x2_v, b0, b1, b2,
           sem1, sem2):
        wid = lax.axis_index("s") * 2 + lax.axis_index("c")
        for c in range(per // CT):
            base = wid * per + c * CT
            pltpu.sync_copy(i1_hbm.at[pl.ds(base, CT)], x1_v)
            pltpu.sync_copy(i2_hbm.at[pl.ds(base, CT)], x2_v)
            cp1 = pltpu.async_copy(ys_hbm.at[x1_v], b1, sem1)
            cp2 = pltpu.async_copy(ys_hbm.at[x2_v], b2, sem2)
            pltpu.sync_copy(sh_hbm.at[pl.ds(base, CT)], b0)
            cp1.wait()
            cp2.wait()

            def cbody(i, carry):
                for r in range(CT):
                    s = pl.ds(i * 16, 16)
                    b0[r, s] = b0[r, s] + b1[r, s] + b2[r, s]
                return carry

            lax.fori_loop(0, D // 16, cbody, 0)
            pltpu.sync_copy(b0, out_hbm.at[pl.ds(base, CT)])

    return ck(shared, ys, inv1, inv2)


def kernel(hidden_states, gate_w, Wg, Wu, Wd, Sg, Su, Sd, seg_w):
    b, s, d = hidden_states.shape
    x = hidden_states.reshape(-1, d)
    T, D = x.shape
    E, DFF, _ = Wg.shape
    DFF_S = Sg.shape[0]
    NP = ((2 * T + E * (BLK - 1)) + CH - 1) // CH * CH
    MAXB = NP // BLK
    MB = (MAXB + 7) // 8 * 8

    gw_pad = jnp.zeros((128, D), F32).at[:E].set(gate_w)
    pos, warr, inv1, inv2, blkexp, nblk, xi = pl.pallas_call(
        _route_body,
        out_shape=[
            jax.ShapeDtypeStruct((2 * T, 1), F32),
            jax.ShapeDtypeStruct((2 * T, 1), F32),
            jax.ShapeDtypeStruct((T, 1), I32),
            jax.ShapeDtypeStruct((T, 1), I32),
            jax.ShapeDtypeStruct((MB, 1), I32),
            jax.ShapeDtypeStruct((1, 1), I32),
            jax.ShapeDtypeStruct((T, D // 2), I32),
        ],
    )(x, gw_pad)

    tok = jnp.tile(jnp.arange(T, dtype=F32), 2).reshape(2 * T, 1)
    perm12, wsort12 = pl.pallas_call(
        _scatter_body,
        grid=(NP // CH,),
        in_specs=[
            pl.BlockSpec((2 * T, 1), lambda q: (0, 0)),
            pl.BlockSpec((2 * T, 1), lambda q: (0, 0)),
            pl.BlockSpec((2 * T, 1), lambda q: (0, 0)),
        ],
        out_specs=[
            pl.BlockSpec((1, 1, CH), lambda q: (q, 0, 0)),
            pl.BlockSpec((1, 1, CH), lambda q: (q, 0, 0)),
        ],
        out_shape=[
            jax.ShapeDtypeStruct((NP // CH, 1, CH), I32),
            jax.ShapeDtypeStruct((NP // CH, 1, CH), F32),
        ],
    )(pos, warr, tok)

    xsi = _sc_gather(xi, perm12.reshape(NP), NP)

    ys = pl.pallas_call(
        _group_body,
        grid_spec=pltpu.PrefetchScalarGridSpec(
            num_scalar_prefetch=2,
            grid=(MAXB,),
            in_specs=[
                pl.BlockSpec((BLK, D // 2), lambda bb, be, nb: (bb, 0)),
                pl.BlockSpec((BLK, 1), lambda bb, be, nb: (bb, 0)),
                pl.BlockSpec((1, DFF, D), lambda bb, be, nb: (be[bb], 0, 0)),
                pl.BlockSpec((1, DFF, D), lambda bb, be, nb: (be[bb], 0, 0)),
                pl.BlockSpec((1, D, DFF), lambda bb, be, nb: (be[bb], 0, 0)),
            ],
            out_specs=pl.BlockSpec((BLK, D), lambda bb, be, nb: (bb, 0)),
        ),
        out_shape=jax.ShapeDtypeStruct((NP, D), F32),
    )(blkexp.reshape(MB), nblk.reshape(1), xsi, wsort12.reshape(NP, 1),
      Wg.astype(BF16), Wu.astype(BF16), Wd.astype(BF16))

    BT = min(512, T)
    BF = 512 if DFF_S % 512 == 0 else DFF_S
    segw_pad = jnp.zeros((128, D), BF16).at[:1].set(seg_w.astype(BF16))
    shared = pl.pallas_call(
        _shared_body,
        grid=(T // BT, DFF_S // BF),
        in_specs=[
            pl.BlockSpec((BT, D), lambda i, j: (i, 0)),
            pl.BlockSpec((BF, D), lambda i, j: (j, 0)),
            pl.BlockSpec((BF, D), lambda i, j: (j, 0)),
            pl.BlockSpec((D, BF), lambda i, j: (0, j)),
            pl.BlockSpec((128, D), lambda i, j: (0, 0)),
        ],
        out_specs=pl.BlockSpec((BT, D), lambda i, j: (i, 0)),
        out_shape=jax.ShapeDtypeStruct((T, D), F32),
    )(x, Sg.astype(BF16), Su.astype(BF16), Sd.astype(BF16), segw_pad)

    out = _sc_combine(shared, ys, inv1.reshape(T), inv2.reshape(T))
    return out.reshape(b, s, d)


# speedup vs baseline: 1.4586x; 1.0011x over previous
import functools

import jax
import jax.numpy as jnp
from jax import lax
from jax.experimental import pallas as pl
from jax.experimental.pallas import tpu as pltpu
from jax.experimental.pallas import tpu_sc as plsc

F32 = jnp.float32
BF16 = jnp.bfloat16
I32 = jnp.int32
_HIGH = lax.Precision.HIGHEST
BLK = 256
CH = 512


def _route_body(x_ref, gw_ref, pos_ref, warr_ref, inv1_ref, inv2_ref,
                blkexp_ref, nblk_ref, xi_ref):
    x = x_ref[...]
    gw = gw_ref[...]
    T = x.shape[0]
    T2 = 2 * T
    D = x.shape[1]
    U32 = jnp.uint32

    def _rne16(v):
        bb = lax.bitcast_convert_type(v, U32)
        return (bb + U32(0x7FFF) + ((bb >> U32(16)) & U32(1))) >> U32(16)

    lo = _rne16(x[:, :D // 2])
    hi = _rne16(x[:, D // 2:])
    xi_ref[...] = lax.bitcast_convert_type(lo | (hi << U32(16)), I32)
    logits = lax.dot_general(x, gw, (((1,), (1,)), ((), ())),
                             preferred_element_type=F32)
    lane = lax.broadcasted_iota(I32, (T, 128), 1)
    logits = jnp.where(lane < 8, logits, -1e30)
    m = jnp.max(logits, axis=1, keepdims=True)
    p = jnp.exp(logits - m)
    p = p / jnp.sum(p, axis=1, keepdims=True)
    w1 = jnp.max(p, axis=1, keepdims=True)
    i1 = jnp.min(jnp.where(p == w1, lane, 999), axis=1, keepdims=True)
    p2 = jnp.where(lane == i1, -1.0, p)
    w2 = jnp.max(p2, axis=1, keepdims=True)
    i2 = jnp.min(jnp.where(p2 == w2, lane, 999), axis=1, keepdims=True)

    e_arr = jnp.concatenate([i1, i2], axis=0)
    w_arr = jnp.concatenate([w1, w2], axis=0)
    lane2 = lax.broadcasted_iota(I32, (T2, 128), 1)
    A = (e_arr == lane2).astype(F32)
    counts = jnp.sum(A, axis=0, keepdims=True)
    cnt_pad = jnp.floor((counts + (BLK - 1)) * (1.0 / BLK)) * BLK
    r128 = lax.broadcasted_iota(I32, (128, 128), 0)
    c128 = lax.broadcasted_iota(I32, (128, 128), 1)
    tri_excl = (r128 < c128).astype(F32)
    offsets = lax.dot_general(cnt_pad, tri_excl, (((1,), (0,)), ((), ())),
                              precision=_HIGH, preferred_element_type=F32)
    rS = lax.broadcasted_iota(I32, (CH, CH), 0)
    cS = lax.broadcasted_iota(I32, (CH, CH), 1)
    tri_strict = (rS > cS).astype(F32)
    run = jnp.zeros((1, 128), F32)
    pos_list = []
    for t in range(T2 // CH):
        At = A[t * CH:(t + 1) * CH]
        within = lax.dot_general(tri_strict, At, (((1,), (0,)), ((), ())),
                                 precision=_HIGH, preferred_element_type=F32)
        pos_list.append(
            jnp.sum(At * (within + run + offsets), axis=1, keepdims=True))
        run = run + jnp.sum(At, axis=0, keepdims=True)
    pos = jnp.concatenate(pos_list, axis=0)
    inv1_ref[...] = pos[:T].astype(I32)
    inv2_ref[...] = pos[T:].astype(I32)

    pos_ref[...] = pos
    warr_ref[...] = w_arr

    MB = blkexp_ref.shape[0]
    rowstart = lax.broadcasted_iota(I32, (MB, 128), 0).astype(F32) * BLK
    indb = ((rowstart >= offsets) & (rowstart < offsets + cnt_pad)
            & (cnt_pad > 0)).astype(F32)
    lanef = lax.broadcasted_iota(I32, (MB, 128), 1).astype(F32)
    be = jnp.sum(indb * lanef, axis=1, keepdims=True)
    validb = jnp.sum(indb, axis=1, keepdims=True)
    lastexp = jnp.max(jnp.where(counts > 0,
                                lax.broadcasted_iota(I32, (1, 128), 1).astype(F32), 0.0),
                      axis=1, keepdims=True)
    blkexp_ref[...] = jnp.where(validb > 0, be, lastexp).astype(I32)
    nblk_ref[...] = (jnp.sum(cnt_pad, axis=1, keepdims=True)
                     * (1.0 / BLK)).astype(I32)


def _scatter_body(pos_ref, warr_ref, tok_ref, perm_ref, wsort_ref):
    q = pl.program_id(0)
    pos = pos_ref[...]
    T2 = pos.shape[0]
    pv = lax.broadcasted_iota(I32, (T2, CH), 1).astype(F32) + (q * CH).astype(
        F32)
    ind = (pos == pv).astype(F32)
    perm_ref[0] = jnp.sum(ind * tok_ref[...], axis=0,
                          keepdims=True).astype(I32)
    wsort_ref[0] = jnp.sum(ind * warr_ref[...], axis=0, keepdims=True)


def _group_body(be_ref, nb_ref, xs_ref, ws_ref, wg_ref, wu_ref, wd_ref,
                out_ref):
    b = pl.program_id(0)

    @pl.when(b < nb_ref[0])
    def _():
        xw = lax.bitcast_convert_type(xs_ref[...], jnp.uint32)
        xlo = lax.bitcast_convert_type(xw << jnp.uint32(16), F32).astype(BF16)
        xhi = lax.bitcast_convert_type(xw & jnp.uint32(0xFFFF0000),
                                       F32).astype(BF16)
        xb = jnp.concatenate([xlo, xhi], axis=1)
        g = lax.dot_general(xb, wg_ref[0], (((1,), (1,)), ((), ())),
                            preferred_element_type=F32)
        u = lax.dot_general(xb, wu_ref[0], (((1,), (1,)), ((), ())),
                            preferred_element_type=F32)
        h = (g * jax.nn.sigmoid(g) * u).astype(BF16)
        y = lax.dot_general(h, wd_ref[0], (((1,), (1,)), ((), ())),
                            preferred_element_type=F32)
        out_ref[...] = y * ws_ref[...]


def _shared_body(x_ref, sg_ref, su_ref, sd_ref, segw_ref, out_ref):
    j = pl.program_id(1)
    nj = pl.num_programs(1)
    x = x_ref[...].astype(BF16)
    g = lax.dot_general(x, sg_ref[...], (((1,), (1,)), ((), ())),
                        preferred_element_type=F32)
    u = lax.dot_general(x, su_ref[...], (((1,), (1,)), ((), ())),
                        preferred_element_type=F32)
    h = (g * jax.nn.sigmoid(g) * u).astype(BF16)
    y = lax.dot_general(h, sd_ref[...], (((1,), (1,)), ((), ())),
                        preferred_element_type=F32)

    @pl.when(j == 0)
    def _init():
        out_ref[...] = y

    @pl.when(j > 0)
    def _acc():
        out_ref[...] += y

    @pl.when(j == nj - 1)
    def _fin():
        sl = lax.dot_general(x, segw_ref[...], (((1,), (1,)), ((), ())),
                             preferred_element_type=F32)
        lane = lax.broadcasted_iota(I32, sl.shape, 1)
        gate = jnp.sum(jnp.where(lane == 0, jax.nn.sigmoid(sl), 0.0),
                       axis=1, keepdims=True)
        out_ref[...] = gate * out_ref[...]


def _sc_gather(x3, perm, NP):
    Tn, W = x3.shape
    NW = 32
    rows_per = NP // NW
    CG = rows_per // 2 if rows_per > 96 else rows_per
    mesh = plsc.VectorSubcoreMesh(core_axis_name="c", subcore_axis_name="s")

    @functools.partial(
        pl.kernel, mesh=mesh,
        out_type=jax.ShapeDtypeStruct((NP, W), I32),
        scratch_types=[
            pltpu.VMEM((CG,), I32),
            pltpu.VMEM((CG, W), I32),
            pltpu.SemaphoreType.DMA,
        ],
    )
    def gk(x_hbm, perm_hbm, out_hbm, idx_v, rows_v, sem):
        wid = lax.axis_index("s") * 2 + lax.axis_index("c")
        for c in range(rows_per // CG):
            base = wid * rows_per + c * CG
            pltpu.sync_copy(perm_hbm.at[pl.ds(base, CG)], idx_v)
            pltpu.async_copy(x_hbm.at[idx_v], rows_v, sem).wait()
            pltpu.sync_copy(rows_v, out_hbm.at[pl.ds(base, CG)])

    return gk(x3, perm)


def _sc_combine(shared, ys, inv1, inv2):
    T, D = shared.shape
    NW = 32
    per = T // NW
    CT = 16
    mesh = plsc.VectorSubcoreMesh(core_axis_name="c", subcore_axis_name="s")

    @functools.partial(
        pl.kernel, mesh=mesh,
        out_type=jax.ShapeDtypeStruct((T, D), F32),
        scratch_types=[
            pltpu.VMEM((CT,), I32),
            pltpu.VMEM((CT,), I32),
            pltpu.VMEM((CT, D), F32),
            pltpu.VMEM((CT, D), F32),
            pltpu.VMEM((CT, D), F32),
            pltpu.SemaphoreType.DMA,
            pltpu.SemaphoreType.DMA,
        ],
    )
    def ck(sh_hbm, ys_hbm, i1_hbm, i2_hbm, out_hbm, x1_v, x2_v, b0, b1, b2,
           sem1, sem2):
        wid = lax.axis_index("s") * 2 + lax.axis_index("c")
        for c in range(per // CT):
            base = wid * per + c * CT
            pltpu.sync_copy(i1_hbm.at[pl.ds(base, CT)], x1_v)
            pltpu.sync_copy(i2_hbm.at[pl.ds(base, CT)], x2_v)
            cp1 = pltpu.async_copy(ys_hbm.at[x1_v], b1, sem1)
            cp2 = pltpu.async_copy(ys_hbm.at[x2_v], b2, sem2)
            pltpu.sync_copy(sh_hbm.at[pl.ds(base, CT)], b0)
            cp1.wait()
            cp2.wait()

            def cbody(i, carry):
                for r in range(CT):
                    s = pl.ds(i * 16, 16)
                    b0[r, s] = b0[r, s] + b1[r, s] + b2[r, s]
                return carry

            lax.fori_loop(0, D // 16, cbody, 0)
            pltpu.sync_copy(b0, out_hbm.at[pl.ds(base, CT)])

    return ck(shared, ys, inv1, inv2)


def kernel(hidden_states, gate_w, Wg, Wu, Wd, Sg, Su, Sd, seg_w):
    b, s, d = hidden_states.shape
    x = hidden_states.reshape(-1, d)
    T, D = x.shape
    E, DFF, _ = Wg.shape
    DFF_S = Sg.shape[0]
    NP = ((2 * T + E * (BLK - 1)) + CH - 1) // CH * CH
    MAXB = NP // BLK
    MB = (MAXB + 7) // 8 * 8

    gw_pad = jnp.zeros((128, D), F32).at[:E].set(gate_w)
    pos, warr, inv1, inv2, blkexp, nblk, xi = pl.pallas_call(
        _route_body,
        out_shape=[
            jax.ShapeDtypeStruct((2 * T, 1), F32),
            jax.ShapeDtypeStruct((2 * T, 1), F32),
            jax.ShapeDtypeStruct((T, 1), I32),
            jax.ShapeDtypeStruct((T, 1), I32),
            jax.ShapeDtypeStruct((MB, 1), I32),
            jax.ShapeDtypeStruct((1, 1), I32),
            jax.ShapeDtypeStruct((T, D // 2), I32),
        ],
    )(x, gw_pad)

    tok = jnp.tile(jnp.arange(T, dtype=F32), 2).reshape(2 * T, 1)
    perm12, wsort12 = pl.pallas_call(
        _scatter_body,
        grid=(NP // CH,),
        in_specs=[
            pl.BlockSpec((2 * T, 1), lambda q: (0, 0)),
            pl.BlockSpec((2 * T, 1), lambda q: (0, 0)),
            pl.BlockSpec((2 * T, 1), lambda q: (0, 0)),
        ],
        out_specs=[
            pl.BlockSpec((1, 1, CH), lambda q: (q, 0, 0)),
            pl.BlockSpec((1, 1, CH), lambda q: (q, 0, 0)),
        ],
        out_shape=[
            jax.ShapeDtypeStruct((NP // CH, 1, CH), I32),
            jax.ShapeDtypeStruct((NP // CH, 1, CH), F32),
        ],
    )(pos, warr, tok)

    xsi = _sc_gather(xi, perm12.reshape(NP), NP)

    BT = min(512, T)
    BF = 512 if DFF_S % 512 == 0 else DFF_S
    segw_pad = jnp.zeros((128, D), BF16).at[:1].set(seg_w.astype(BF16))
    shared = pl.pallas_call(
        _shared_body,
        grid=(T // BT, DFF_S // BF),
        in_specs=[
            pl.BlockSpec((BT, D), lambda i, j: (i, 0)),
            pl.BlockSpec((BF, D), lambda i, j: (j, 0)),
            pl.BlockSpec((BF, D), lambda i, j: (j, 0)),
            pl.BlockSpec((D, BF), lambda i, j: (0, j)),
            pl.BlockSpec((128, D), lambda i, j: (0, 0)),
        ],
        out_specs=pl.BlockSpec((BT, D), lambda i, j: (i, 0)),
        out_shape=jax.ShapeDtypeStruct((T, D), F32),
    )(x, Sg.astype(BF16), Su.astype(BF16), Sd.astype(BF16), segw_pad)

    ys = pl.pallas_call(
        _group_body,
        grid_spec=pltpu.PrefetchScalarGridSpec(
            num_scalar_prefetch=2,
            grid=(MAXB,),
            in_specs=[
                pl.BlockSpec((BLK, D // 2), lambda bb, be, nb: (bb, 0)),
                pl.BlockSpec((BLK, 1), lambda bb, be, nb: (bb, 0)),
                pl.BlockSpec((1, DFF, D), lambda bb, be, nb: (be[bb], 0, 0)),
                pl.BlockSpec((1, DFF, D), lambda bb, be, nb: (be[bb], 0, 0)),
                pl.BlockSpec((1, D, DFF), lambda bb, be, nb: (be[bb], 0, 0)),
            ],
            out_specs=pl.BlockSpec((BLK, D), lambda bb, be, nb: (bb, 0)),
        ),
        out_shape=jax.ShapeDtypeStruct((NP, D), F32),
    )(blkexp.reshape(MB), nblk.reshape(1), xsi, wsort12.reshape(NP, 1),
      Wg.astype(BF16), Wu.astype(BF16), Wd.astype(BF16))


    out = _sc_combine(shared, ys, inv1.reshape(T), inv2.reshape(T))
    return out.reshape(b, s, d)


# speedup vs baseline: 1.6549x; 1.1346x over previous
import functools

import jax
import jax.numpy as jnp
from jax import lax
from jax.experimental import pallas as pl
from jax.experimental.pallas import tpu as pltpu
from jax.experimental.pallas import tpu_sc as plsc

F32 = jnp.float32
BF16 = jnp.bfloat16
I32 = jnp.int32
_HIGH = lax.Precision.HIGHEST
BLK = 256
CH = 512


def _route_body(x_ref, gw_ref, pos_ref, warr_ref, inv1_ref, inv2_ref,
                blkexp_ref, nblk_ref, xi_ref):
    x = x_ref[...]
    gw = gw_ref[...]
    T = x.shape[0]
    T2 = 2 * T
    xi_ref[...] = x.astype(BF16)
    logits = lax.dot_general(x, gw, (((1,), (1,)), ((), ())),
                             preferred_element_type=F32)
    lane = lax.broadcasted_iota(I32, (T, 128), 1)
    logits = jnp.where(lane < 8, logits, -1e30)
    m = jnp.max(logits, axis=1, keepdims=True)
    p = jnp.exp(logits - m)
    p = p / jnp.sum(p, axis=1, keepdims=True)
    w1 = jnp.max(p, axis=1, keepdims=True)
    i1 = jnp.min(jnp.where(p == w1, lane, 999), axis=1, keepdims=True)
    p2 = jnp.where(lane == i1, -1.0, p)
    w2 = jnp.max(p2, axis=1, keepdims=True)
    i2 = jnp.min(jnp.where(p2 == w2, lane, 999), axis=1, keepdims=True)

    e_arr = jnp.concatenate([i1, i2], axis=0)
    w_arr = jnp.concatenate([w1, w2], axis=0)
    lane2 = lax.broadcasted_iota(I32, (T2, 128), 1)
    A = (e_arr == lane2).astype(F32)
    counts = jnp.sum(A, axis=0, keepdims=True)
    cnt_pad = jnp.floor((counts + (BLK - 1)) * (1.0 / BLK)) * BLK
    r128 = lax.broadcasted_iota(I32, (128, 128), 0)
    c128 = lax.broadcasted_iota(I32, (128, 128), 1)
    tri_excl = (r128 < c128).astype(F32)
    offsets = lax.dot_general(cnt_pad, tri_excl, (((1,), (0,)), ((), ())),
                              precision=_HIGH, preferred_element_type=F32)
    rS = lax.broadcasted_iota(I32, (CH, CH), 0)
    cS = lax.broadcasted_iota(I32, (CH, CH), 1)
    tri_strict = (rS > cS).astype(F32)
    run = jnp.zeros((1, 128), F32)
    pos_list = []
    for t in range(T2 // CH):
        At = A[t * CH:(t + 1) * CH]
        within = lax.dot_general(tri_strict, At, (((1,), (0,)), ((), ())),
                                 precision=_HIGH, preferred_element_type=F32)
        pos_list.append(
            jnp.sum(At * (within + run + offsets), axis=1, keepdims=True))
        run = run + jnp.sum(At, axis=0, keepdims=True)
    pos = jnp.concatenate(pos_list, axis=0)
    inv1_ref[...] = pos[:T].astype(I32)
    inv2_ref[...] = pos[T:].astype(I32)

    pos_ref[...] = pos
    warr_ref[...] = w_arr

    MB = blkexp_ref.shape[0]
    rowstart = lax.broadcasted_iota(I32, (MB, 128), 0).astype(F32) * BLK
    indb = ((rowstart >= offsets) & (rowstart < offsets + cnt_pad)
            & (cnt_pad > 0)).astype(F32)
    lanef = lax.broadcasted_iota(I32, (MB, 128), 1).astype(F32)
    be = jnp.sum(indb * lanef, axis=1, keepdims=True)
    validb = jnp.sum(indb, axis=1, keepdims=True)
    lastexp = jnp.max(jnp.where(counts > 0,
                                lax.broadcasted_iota(I32, (1, 128), 1).astype(F32), 0.0),
                      axis=1, keepdims=True)
    blkexp_ref[...] = jnp.where(validb > 0, be, lastexp).astype(I32)
    nblk_ref[...] = (jnp.sum(cnt_pad, axis=1, keepdims=True)
                     * (1.0 / BLK)).astype(I32)


def _scatter_body(pos_ref, warr_ref, tok_ref, perm_ref, wsort_ref):
    q = pl.program_id(0)
    pos = pos_ref[...]
    T2 = pos.shape[0]
    pv = lax.broadcasted_iota(I32, (T2, CH), 1).astype(F32) + (q * CH).astype(
        F32)
    ind = (pos == pv).astype(F32)
    perm_ref[0] = jnp.sum(ind * tok_ref[...], axis=0,
                          keepdims=True).astype(I32)
    wsort_ref[0] = jnp.sum(ind * warr_ref[...], axis=0, keepdims=True)


def _group_body(be_ref, nb_ref, xbf_ref, pm_ref, ws_ref, wg_ref, wu_ref,
                wd_ref, out_ref):
    b = pl.program_id(0)

    @pl.when(b < nb_ref[0])
    def _():
        T = xbf_ref.shape[0]
        tokl = lax.broadcasted_iota(I32, (pm_ref.shape[0], T), 1)
        P = (pm_ref[...] == tokl).astype(BF16)
        xb = lax.dot_general(P, xbf_ref[...], (((1,), (0,)), ((), ())),
                             preferred_element_type=F32).astype(BF16)
        g = lax.dot_general(xb, wg_ref[0], (((1,), (1,)), ((), ())),
                            preferred_element_type=F32)
        u = lax.dot_general(xb, wu_ref[0], (((1,), (1,)), ((), ())),
                            preferred_element_type=F32)
        h = (g * jax.nn.sigmoid(g) * u).astype(BF16)
        y = lax.dot_general(h, wd_ref[0], (((1,), (1,)), ((), ())),
                            preferred_element_type=F32)
        out_ref[...] = y * ws_ref[...]


def _shared_body(x_ref, sg_ref, su_ref, sd_ref, segw_ref, out_ref):
    j = pl.program_id(1)
    nj = pl.num_programs(1)
    x = x_ref[...].astype(BF16)
    g = lax.dot_general(x, sg_ref[...], (((1,), (1,)), ((), ())),
                        preferred_element_type=F32)
    u = lax.dot_general(x, su_ref[...], (((1,), (1,)), ((), ())),
                        preferred_element_type=F32)
    h = (g * jax.nn.sigmoid(g) * u).astype(BF16)
    y = lax.dot_general(h, sd_ref[...], (((1,), (1,)), ((), ())),
                        preferred_element_type=F32)

    @pl.when(j == 0)
    def _init():
        out_ref[...] = y

    @pl.when(j > 0)
    def _acc():
        out_ref[...] += y

    @pl.when(j == nj - 1)
    def _fin():
        sl = lax.dot_general(x, segw_ref[...], (((1,), (1,)), ((), ())),
                             preferred_element_type=F32)
        lane = lax.broadcasted_iota(I32, sl.shape, 1)
        gate = jnp.sum(jnp.where(lane == 0, jax.nn.sigmoid(sl), 0.0),
                       axis=1, keepdims=True)
        out_ref[...] = gate * out_ref[...]


def _sc_gather(x3, perm, NP):
    Tn, W = x3.shape
    NW = 32
    rows_per = NP // NW
    CG = rows_per // 2 if rows_per > 96 else rows_per
    mesh = plsc.VectorSubcoreMesh(core_axis_name="c", subcore_axis_name="s")

    @functools.partial(
        pl.kernel, mesh=mesh,
        out_type=jax.ShapeDtypeStruct((NP, W), I32),
        scratch_types=[
            pltpu.VMEM((CG,), I32),
            pltpu.VMEM((CG, W), I32),
            pltpu.SemaphoreType.DMA,
        ],
    )
    def gk(x_hbm, perm_hbm, out_hbm, idx_v, rows_v, sem):
        wid = lax.axis_index("s") * 2 + lax.axis_index("c")
        for c in range(rows_per // CG):
            base = wid * rows_per + c * CG
            pltpu.sync_copy(perm_hbm.at[pl.ds(base, CG)], idx_v)
            pltpu.async_copy(x_hbm.at[idx_v], rows_v, sem).wait()
            pltpu.sync_copy(rows_v, out_hbm.at[pl.ds(base, CG)])

    return gk(x3, perm)


def _sc_combine(shared, ys, inv1, inv2):
    T, D = shared.shape
    NW = 32
    per = T // NW
    CT = 16
    mesh = plsc.VectorSubcoreMesh(core_axis_name="c", subcore_axis_name="s")

    @functools.partial(
        pl.kernel, mesh=mesh,
        out_type=jax.ShapeDtypeStruct((T, D), F32),
        scratch_types=[
            pltpu.VMEM((CT,), I32),
            pltpu.VMEM((CT,), I32),
            pltpu.VMEM((CT, D), F32),
            pltpu.VMEM((CT, D), F32),
            pltpu.VMEM((CT, D), F32),
            pltpu.SemaphoreType.DMA,
            pltpu.SemaphoreType.DMA,
        ],
    )
    def ck(sh_hbm, ys_hbm, i1_hbm, i2_hbm, out_hbm, x1_v, x2_v, b0, b1, b2,
           sem1, sem2):
        wid = lax.axis_index("s") * 2 + lax.axis_index("c")
        for c in range(per // CT):
            base = wid * per + c * CT
            pltpu.sync_copy(i1_hbm.at[pl.ds(base, CT)], x1_v)
            pltpu.sync_copy(i2_hbm.at[pl.ds(base, CT)], x2_v)
            cp1 = pltpu.async_copy(ys_hbm.at[x1_v], b1, sem1)
            cp2 = pltpu.async_copy(ys_hbm.at[x2_v], b2, sem2)
            pltpu.sync_copy(sh_hbm.at[pl.ds(base, CT)], b0)
            cp1.wait()
            cp2.wait()

            def cbody(i, carry):
                for r in range(CT):
                    s = pl.ds(i * 16, 16)
                    b0[r, s] = b0[r, s] + b1[r, s] + b2[r, s]
                return carry

            lax.fori_loop(0, D // 16, cbody, 0)
            pltpu.sync_copy(b0, out_hbm.at[pl.ds(base, CT)])

    return ck(shared, ys, inv1, inv2)


def kernel(hidden_states, gate_w, Wg, Wu, Wd, Sg, Su, Sd, seg_w):
    b, s, d = hidden_states.shape
    x = hidden_states.reshape(-1, d)
    T, D = x.shape
    E, DFF, _ = Wg.shape
    DFF_S = Sg.shape[0]
    NP = ((2 * T + E * (BLK - 1)) + CH - 1) // CH * CH
    MAXB = NP // BLK
    MB = (MAXB + 7) // 8 * 8

    gw_pad = jnp.zeros((128, D), F32).at[:E].set(gate_w)
    pos, warr, inv1, inv2, blkexp, nblk, xbf = pl.pallas_call(
        _route_body,
        out_shape=[
            jax.ShapeDtypeStruct((2 * T, 1), F32),
            jax.ShapeDtypeStruct((2 * T, 1), F32),
            jax.ShapeDtypeStruct((T, 1), I32),
            jax.ShapeDtypeStruct((T, 1), I32),
            jax.ShapeDtypeStruct((MB, 1), I32),
            jax.ShapeDtypeStruct((1, 1), I32),
            jax.ShapeDtypeStruct((T, D), BF16),
        ],
    )(x, gw_pad)

    tok = jnp.tile(jnp.arange(T, dtype=F32), 2).reshape(2 * T, 1)
    perm12, wsort12 = pl.pallas_call(
        _scatter_body,
        grid=(NP // CH,),
        in_specs=[
            pl.BlockSpec((2 * T, 1), lambda q: (0, 0)),
            pl.BlockSpec((2 * T, 1), lambda q: (0, 0)),
            pl.BlockSpec((2 * T, 1), lambda q: (0, 0)),
        ],
        out_specs=[
            pl.BlockSpec((1, 1, CH), lambda q: (q, 0, 0)),
            pl.BlockSpec((1, 1, CH), lambda q: (q, 0, 0)),
        ],
        out_shape=[
            jax.ShapeDtypeStruct((NP // CH, 1, CH), I32),
            jax.ShapeDtypeStruct((NP // CH, 1, CH), F32),
        ],
    )(pos, warr, tok)

    BT = min(512, T)
    BF = 512 if DFF_S % 512 == 0 else DFF_S
    segw_pad = jnp.zeros((128, D), BF16).at[:1].set(seg_w.astype(BF16))
    shared = pl.pallas_call(
        _shared_body,
        grid=(T // BT, DFF_S // BF),
        in_specs=[
            pl.BlockSpec((BT, D), lambda i, j: (i, 0)),
            pl.BlockSpec((BF, D), lambda i, j: (j, 0)),
            pl.BlockSpec((BF, D), lambda i, j: (j, 0)),
            pl.BlockSpec((D, BF), lambda i, j: (0, j)),
            pl.BlockSpec((128, D), lambda i, j: (0, 0)),
        ],
        out_specs=pl.BlockSpec((BT, D), lambda i, j: (i, 0)),
        out_shape=jax.ShapeDtypeStruct((T, D), F32),
    )(x, Sg.astype(BF16), Su.astype(BF16), Sd.astype(BF16), segw_pad)

    ys = pl.pallas_call(
        _group_body,
        grid_spec=pltpu.PrefetchScalarGridSpec(
            num_scalar_prefetch=2,
            grid=(MAXB,),
            in_specs=[
                pl.BlockSpec((T, D), lambda bb, be, nb: (0, 0)),
                pl.BlockSpec((BLK, 1), lambda bb, be, nb: (bb, 0)),
                pl.BlockSpec((BLK, 1), lambda bb, be, nb: (bb, 0)),
                pl.BlockSpec((1, DFF, D), lambda bb, be, nb: (be[bb], 0, 0)),
                pl.BlockSpec((1, DFF, D), lambda bb, be, nb: (be[bb], 0, 0)),
                pl.BlockSpec((1, D, DFF), lambda bb, be, nb: (be[bb], 0, 0)),
            ],
            out_specs=pl.BlockSpec((BLK, D), lambda bb, be, nb: (bb, 0)),
        ),
        out_shape=jax.ShapeDtypeStruct((NP, D), F32),
    )(blkexp.reshape(MB), nblk.reshape(1), xbf, perm12.reshape(NP, 1),
      wsort12.reshape(NP, 1), Wg.astype(BF16), Wu.astype(BF16),
      Wd.astype(BF16))


    out = _sc_combine(shared, ys, inv1.reshape(T), inv2.reshape(T))
    return out.reshape(b, s, d)


# speedup vs baseline: 1.7866x; 1.0795x over previous
import functools

import jax
import jax.numpy as jnp
from jax import lax
from jax.experimental import pallas as pl
from jax.experimental.pallas import tpu as pltpu
from jax.experimental.pallas import tpu_sc as plsc

F32 = jnp.float32
BF16 = jnp.bfloat16
I32 = jnp.int32
_HIGH = lax.Precision.HIGHEST
BLK = 256
CH = 512


def _route_body(x_ref, gw_ref, pos_ref, warr_ref, inv1_ref, inv2_ref,
                blkexp_ref, nblk_ref, xi_ref):
    x = x_ref[...]
    gw = gw_ref[...]
    T = x.shape[0]
    T2 = 2 * T
    xi_ref[...] = x.astype(BF16)
    logits = lax.dot_general(x, gw, (((1,), (1,)), ((), ())),
                             preferred_element_type=F32)
    lane = lax.broadcasted_iota(I32, (T, 128), 1)
    logits = jnp.where(lane < 8, logits, -1e30)
    m = jnp.max(logits, axis=1, keepdims=True)
    p = jnp.exp(logits - m)
    p = p / jnp.sum(p, axis=1, keepdims=True)
    w1 = jnp.max(p, axis=1, keepdims=True)
    i1 = jnp.min(jnp.where(p == w1, lane, 999), axis=1, keepdims=True)
    p2 = jnp.where(lane == i1, -1.0, p)
    w2 = jnp.max(p2, axis=1, keepdims=True)
    i2 = jnp.min(jnp.where(p2 == w2, lane, 999), axis=1, keepdims=True)

    e_arr = jnp.concatenate([i1, i2], axis=0)
    w_arr = jnp.concatenate([w1, w2], axis=0)
    lane2 = lax.broadcasted_iota(I32, (T2, 128), 1)
    A = (e_arr == lane2).astype(F32)
    counts = jnp.sum(A, axis=0, keepdims=True)
    cnt_pad = jnp.floor((counts + (BLK - 1)) * (1.0 / BLK)) * BLK
    r128 = lax.broadcasted_iota(I32, (128, 128), 0)
    c128 = lax.broadcasted_iota(I32, (128, 128), 1)
    tri_excl = (r128 < c128).astype(F32)
    offsets = lax.dot_general(cnt_pad, tri_excl, (((1,), (0,)), ((), ())),
                              precision=_HIGH, preferred_element_type=F32)
    rS = lax.broadcasted_iota(I32, (CH, CH), 0)
    cS = lax.broadcasted_iota(I32, (CH, CH), 1)
    tri_strict = (rS > cS).astype(F32)
    run = jnp.zeros((1, 128), F32)
    pos_list = []
    for t in range(T2 // CH):
        At = A[t * CH:(t + 1) * CH]
        within = lax.dot_general(tri_strict, At, (((1,), (0,)), ((), ())),
                                 precision=_HIGH, preferred_element_type=F32)
        pos_list.append(
            jnp.sum(At * (within + run + offsets), axis=1, keepdims=True))
        run = run + jnp.sum(At, axis=0, keepdims=True)
    pos = jnp.concatenate(pos_list, axis=0)
    inv1_ref[...] = pos[:T].astype(I32)
    inv2_ref[...] = pos[T:].astype(I32)

    pos_ref[...] = pos
    warr_ref[...] = w_arr

    MB = blkexp_ref.shape[0]
    rowstart = lax.broadcasted_iota(I32, (MB, 128), 0).astype(F32) * BLK
    indb = ((rowstart >= offsets) & (rowstart < offsets + cnt_pad)
            & (cnt_pad > 0)).astype(F32)
    lanef = lax.broadcasted_iota(I32, (MB, 128), 1).astype(F32)
    be = jnp.sum(indb * lanef, axis=1, keepdims=True)
    validb = jnp.sum(indb, axis=1, keepdims=True)
    lastexp = jnp.max(jnp.where(counts > 0,
                                lax.broadcasted_iota(I32, (1, 128), 1).astype(F32), 0.0),
                      axis=1, keepdims=True)
    blkexp_ref[...] = jnp.where(validb > 0, be, lastexp).astype(I32)
    nblk_ref[...] = (jnp.sum(cnt_pad, axis=1, keepdims=True)
                     * (1.0 / BLK)).astype(I32)


def _scatter_body(pos_ref, warr_ref, tok_ref, perm_ref, wsort_ref):
    q = pl.program_id(0)
    pos = pos_ref[...]
    T2 = pos.shape[0]
    pv = lax.broadcasted_iota(I32, (T2, CH), 1).astype(F32) + (q * CH).astype(
        F32)
    ind = (pos == pv).astype(F32)
    perm_ref[0] = jnp.sum(ind * tok_ref[...], axis=0,
                          keepdims=True).astype(I32)
    wsort_ref[0] = jnp.sum(ind * warr_ref[...], axis=0, keepdims=True)


def _group_body(be_ref, nb_ref, xbf_ref, pm_ref, ws_ref, wg_ref, wu_ref,
                wd_ref, out_ref):
    b = pl.program_id(0)

    @pl.when(b < nb_ref[0])
    def _():
        T = xbf_ref.shape[0]
        tokl = lax.broadcasted_iota(I32, (pm_ref.shape[0], T), 1)
        P = (pm_ref[...] == tokl).astype(BF16)
        xb = lax.dot_general(P, xbf_ref[...], (((1,), (0,)), ((), ())),
                             preferred_element_type=F32).astype(BF16)
        g = lax.dot_general(xb, wg_ref[0], (((1,), (1,)), ((), ())),
                            preferred_element_type=F32)
        u = lax.dot_general(xb, wu_ref[0], (((1,), (1,)), ((), ())),
                            preferred_element_type=F32)
        h = (g * jax.nn.sigmoid(g) * u).astype(BF16)
        y = lax.dot_general(h, wd_ref[0], (((1,), (1,)), ((), ())),
                            preferred_element_type=F32)
        out_ref[...] = y * ws_ref[...]


def _shared_body(x_ref, sg_ref, su_ref, sd_ref, segw_ref, out_ref):
    j = pl.program_id(1)
    nj = pl.num_programs(1)
    x = x_ref[...]
    g = lax.dot_general(x, sg_ref[...].astype(BF16), (((1,), (1,)), ((), ())),
                        preferred_element_type=F32)
    u = lax.dot_general(x, su_ref[...].astype(BF16), (((1,), (1,)), ((), ())),
                        preferred_element_type=F32)
    h = (g * jax.nn.sigmoid(g) * u).astype(BF16)
    y = lax.dot_general(h, sd_ref[...].astype(BF16), (((1,), (1,)), ((), ())),
                        preferred_element_type=F32)

    @pl.when(j == 0)
    def _init():
        out_ref[...] = y

    @pl.when(j > 0)
    def _acc():
        out_ref[...] += y

    @pl.when(j == nj - 1)
    def _fin():
        sl = lax.dot_general(x, segw_ref[...], (((1,), (1,)), ((), ())),
                             preferred_element_type=F32)
        lane = lax.broadcasted_iota(I32, sl.shape, 1)
        gate = jnp.sum(jnp.where(lane == 0, jax.nn.sigmoid(sl), 0.0),
                       axis=1, keepdims=True)
        out_ref[...] = gate * out_ref[...]


def _sc_gather(x3, perm, NP):
    Tn, W = x3.shape
    NW = 32
    rows_per = NP // NW
    CG = rows_per // 2 if rows_per > 96 else rows_per
    mesh = plsc.VectorSubcoreMesh(core_axis_name="c", subcore_axis_name="s")

    @functools.partial(
        pl.kernel, mesh=mesh,
        out_type=jax.ShapeDtypeStruct((NP, W), I32),
        scratch_types=[
            pltpu.VMEM((CG,), I32),
            pltpu.VMEM((CG, W), I32),
            pltpu.SemaphoreType.DMA,
        ],
    )
    def gk(x_hbm, perm_hbm, out_hbm, idx_v, rows_v, sem):
        wid = lax.axis_index("s") * 2 + lax.axis_index("c")
        for c in range(rows_per // CG):
            base = wid * rows_per + c * CG
            pltpu.sync_copy(perm_hbm.at[pl.ds(base, CG)], idx_v)
            pltpu.async_copy(x_hbm.at[idx_v], rows_v, sem).wait()
            pltpu.sync_copy(rows_v, out_hbm.at[pl.ds(base, CG)])

    return gk(x3, perm)


def _sc_combine(shared, ys, inv1, inv2):
    T, D = shared.shape
    NW = 32
    per = T // NW
    CT = 16
    mesh = plsc.VectorSubcoreMesh(core_axis_name="c", subcore_axis_name="s")

    @functools.partial(
        pl.kernel, mesh=mesh,
        out_type=jax.ShapeDtypeStruct((T, D), F32),
        scratch_types=[
            pltpu.VMEM((CT,), I32),
            pltpu.VMEM((CT,), I32),
            pltpu.VMEM((CT, D), F32),
            pltpu.VMEM((CT, D), F32),
            pltpu.VMEM((CT, D), F32),
            pltpu.SemaphoreType.DMA,
            pltpu.SemaphoreType.DMA,
        ],
    )
    def ck(sh_hbm, ys_hbm, i1_hbm, i2_hbm, out_hbm, x1_v, x2_v, b0, b1, b2,
           sem1, sem2):
        wid = lax.axis_index("s") * 2 + lax.axis_index("c")
        for c in range(per // CT):
            base = wid * per + c * CT
            pltpu.sync_copy(i1_hbm.at[pl.ds(base, CT)], x1_v)
            pltpu.sync_copy(i2_hbm.at[pl.ds(base, CT)], x2_v)
            cp1 = pltpu.async_copy(ys_hbm.at[x1_v], b1, sem1)
            cp2 = pltpu.async_copy(ys_hbm.at[x2_v], b2, sem2)
            pltpu.sync_copy(sh_hbm.at[pl.ds(base, CT)], b0)
            cp1.wait()
            cp2.wait()

            def cbody(i, carry):
                for r in range(CT):
                    s = pl.ds(i * 16, 16)
                    b0[r, s] = b0[r, s] + b1[r, s] + b2[r, s]
                return carry

            lax.fori_loop(0, D // 16, cbody, 0)
            pltpu.sync_copy(b0, out_hbm.at[pl.ds(base, CT)])

    return ck(shared, ys, inv1, inv2)


def kernel(hidden_states, gate_w, Wg, Wu, Wd, Sg, Su, Sd, seg_w):
    b, s, d = hidden_states.shape
    x = hidden_states.reshape(-1, d)
    T, D = x.shape
    E, DFF, _ = Wg.shape
    DFF_S = Sg.shape[0]
    NP = ((2 * T + E * (BLK - 1)) + CH - 1) // CH * CH
    MAXB = NP // BLK
    MB = (MAXB + 7) // 8 * 8

    gw_pad = jnp.zeros((128, D), F32).at[:E].set(gate_w)
    pos, warr, inv1, inv2, blkexp, nblk, xbf = pl.pallas_call(
        _route_body,
        out_shape=[
            jax.ShapeDtypeStruct((2 * T, 1), F32),
            jax.ShapeDtypeStruct((2 * T, 1), F32),
            jax.ShapeDtypeStruct((T, 1), I32),
            jax.ShapeDtypeStruct((T, 1), I32),
            jax.ShapeDtypeStruct((MB, 1), I32),
            jax.ShapeDtypeStruct((1, 1), I32),
            jax.ShapeDtypeStruct((T, D), BF16),
        ],
    )(x, gw_pad)

    tok = jnp.tile(jnp.arange(T, dtype=F32), 2).reshape(2 * T, 1)
    perm12, wsort12 = pl.pallas_call(
        _scatter_body,
        grid=(NP // CH,),
        in_specs=[
            pl.BlockSpec((2 * T, 1), lambda q: (0, 0)),
            pl.BlockSpec((2 * T, 1), lambda q: (0, 0)),
            pl.BlockSpec((2 * T, 1), lambda q: (0, 0)),
        ],
        out_specs=[
            pl.BlockSpec((1, 1, CH), lambda q: (q, 0, 0)),
            pl.BlockSpec((1, 1, CH), lambda q: (q, 0, 0)),
        ],
        out_shape=[
            jax.ShapeDtypeStruct((NP // CH, 1, CH), I32),
            jax.ShapeDtypeStruct((NP // CH, 1, CH), F32),
        ],
    )(pos, warr, tok)

    BT = min(512, T)
    BF = 512 if DFF_S % 512 == 0 else DFF_S
    segw_pad = jnp.zeros((128, D), BF16).at[:1].set(seg_w.astype(BF16))
    shared = pl.pallas_call(
        _shared_body,
        grid=(T // BT, DFF_S // BF),
        in_specs=[
            pl.BlockSpec((BT, D), lambda i, j: (i, 0)),
            pl.BlockSpec((BF, D), lambda i, j: (j, 0)),
            pl.BlockSpec((BF, D), lambda i, j: (j, 0)),
            pl.BlockSpec((D, BF), lambda i, j: (0, j)),
            pl.BlockSpec((128, D), lambda i, j: (0, 0)),
        ],
        out_specs=pl.BlockSpec((BT, D), lambda i, j: (i, 0)),
        out_shape=jax.ShapeDtypeStruct((T, D), F32),
    )(xbf, Sg, Su, Sd, segw_pad)

    ys = pl.pallas_call(
        _group_body,
        grid_spec=pltpu.PrefetchScalarGridSpec(
            num_scalar_prefetch=2,
            grid=(MAXB,),
            in_specs=[
                pl.BlockSpec((T, D), lambda bb, be, nb: (0, 0)),
                pl.BlockSpec((BLK, 1), lambda bb, be, nb: (bb, 0)),
                pl.BlockSpec((BLK, 1), lambda bb, be, nb: (bb, 0)),
                pl.BlockSpec((1, DFF, D), lambda bb, be, nb: (be[bb], 0, 0)),
                pl.BlockSpec((1, DFF, D), lambda bb, be, nb: (be[bb], 0, 0)),
                pl.BlockSpec((1, D, DFF), lambda bb, be, nb: (be[bb], 0, 0)),
            ],
            out_specs=pl.BlockSpec((BLK, D), lambda bb, be, nb: (bb, 0)),
        ),
        out_shape=jax.ShapeDtypeStruct((NP, D), F32),
    )(blkexp.reshape(MB), nblk.reshape(1), xbf, perm12.reshape(NP, 1),
      wsort12.reshape(NP, 1), Wg.astype(BF16), Wu.astype(BF16),
      Wd.astype(BF16))


    out = _sc_combine(shared, ys, inv1.reshape(T), inv2.reshape(T))
    return out.reshape(b, s, d)
